# Initial kernel scaffold; baseline (speedup 1.0000x reference)
import jax
import jax.numpy as jnp
from jax.experimental import pallas as pl


def kernel(idx, table):
    raise NotImplementedError("write your pallas kernel here")


# speedup vs baseline: 1.0355x; 1.0355x over previous
import functools

import jax
import jax.numpy as jnp
from jax import lax
from jax.experimental import pallas as pl
from jax.experimental.pallas import tpu as pltpu
from jax.experimental.pallas import tpu_sc as plsc

_VOCAB = 1000
_D = 1000
_B = 1024
_T = 50
_NTOK = _B * _T

_NC = 2
_NS = 16
_NW = _NC * _NS
_TPW = _NTOK // _NW

_CH = 32
_NCH = _TPW // _CH
_NBUF = 2


def _body(table_hbm, idx_hbm, out_hbm, idx_v, bufs, g0, g1, s0, s1):
    wid = lax.axis_index("s") * _NC + lax.axis_index("c")
    base = wid * _TPW
    pltpu.sync_copy(idx_hbm.at[pl.ds(base, _TPW)], idx_v)

    gsem = (g0, g1)
    ssem = (s0, s1)

    def gather_dma(c, b):
        return pltpu.make_async_copy(
            table_hbm.at[idx_v.at[pl.ds(pl.multiple_of(c * _CH, 8), _CH)]],
            bufs.at[b],
            gsem[b],
        )

    def scatter_dma(c, b):
        return pltpu.make_async_copy(
            bufs.at[b],
            out_hbm.at[pl.ds(base + c * _CH, _CH)],
            ssem[b],
        )

    for b in range(_NBUF):
        gather_dma(b, b).start()

    def outer(i, carry):
        cc = i * _NBUF
        for b in range(_NBUF):
            c = cc + b
            gather_dma(c, b).wait()
            scatter_dma(c, b).start()
            scatter_dma(c, b).wait()

            @pl.when(c + _NBUF < _NCH)
            def _():
                gather_dma(c + _NBUF, b).start()

        return carry

    lax.fori_loop(0, _NCH // _NBUF, outer, 0)


@functools.partial(
    pl.kernel,
    mesh=plsc.VectorSubcoreMesh(core_axis_name="c", subcore_axis_name="s"),
    compiler_params=pltpu.CompilerParams(use_tc_tiling_on_sc=False),
    out_type=jax.ShapeDtypeStruct((_NTOK, _D), jnp.float32),
    scratch_types=[
        pltpu.VMEM((_TPW,), jnp.int32),
        pltpu.VMEM((_NBUF, _CH, _D), jnp.float32),
        pltpu.SemaphoreType.DMA,
        pltpu.SemaphoreType.DMA,
        pltpu.SemaphoreType.DMA,
        pltpu.SemaphoreType.DMA,
    ],
)
def _gather_rows(table_hbm, idx_hbm, out_hbm, idx_v, bufs, g0, g1, s0, s1):
    _body(table_hbm, idx_hbm, out_hbm, idx_v, bufs, g0, g1, s0, s1)


def kernel(idx, table):
    flat = idx.reshape(_NTOK)
    out = _gather_rows(table, flat)
    return out.reshape(_B, _T, _D)
